# baseline (speedup 1.0000x reference)
import functools

import jax
import jax.numpy as jnp
from jax import lax
from jax.experimental import pallas as pl
from jax.experimental.pallas import tpu as pltpu
from jax.experimental.pallas import tpu_sc as plsc

_D = 64
_NC = 2
_NS = 16
_NW = _NC * _NS
_NBUF = 4


def kernel(batch, table):
    b = batch.shape[0]
    per_w = b // _NW
    idx = batch.astype(jnp.int32).reshape(_NW, per_w)

    tt = table.T

    mesh = plsc.VectorSubcoreMesh(core_axis_name="c", subcore_axis_name="s")
    cp = pltpu.CompilerParams(
        needs_layout_passes=False, disable_bounds_checks=True
    )

    @functools.partial(
        pl.kernel,
        mesh=mesh,
        compiler_params=cp,
        out_type=jax.ShapeDtypeStruct((_D, b), table.dtype),
        scratch_types=[
            pltpu.VMEM((per_w,), jnp.int32),
            pltpu.VMEM((_NBUF, _D, 128), jnp.float32),
            pltpu.VMEM((_D, per_w // 2), jnp.float32),
            pltpu.SemaphoreType.DMA((_NBUF,)),
        ],
    )
    def gather_kernel(tt_hbm, idx_hbm, out_hbm, idx_v, tiles_v, out_v, sems):
        wid = lax.axis_index("s") * _NC + lax.axis_index("c")
        pltpu.sync_copy(idx_hbm.at[wid], idx_v)

        iota16 = lax.iota(jnp.int32, 16)
        half_w = per_w // 2

        def fetch(r, slot):
            c128 = pl.multiple_of((r >> 7) * 128, 128)
            pltpu.async_copy(
                tt_hbm.at[:, pl.ds(c128, 128)],
                tiles_v.at[slot],
                sems.at[slot],
            )

        def wait(slot):
            pltpu.make_async_copy(
                tt_hbm.at[:, pl.ds(0, 128)],
                tiles_v.at[slot],
                sems.at[slot],
            ).wait()

        for half in range(2):
            base = half * half_w
            jv0 = idx_v[pl.ds(base, 16)]
            for t in range(_NBUF):
                fetch(jv0[t], t)

            @pl.loop(0, half_w, step=16)
            def _(j0, base=base):
                jv = idx_v[pl.ds(base + j0, 16)]
                nxt = jnp.minimum(base + j0 + 16, per_w - 16)
                jn = idx_v[pl.ds(nxt, 16)]
                for i in range(16):
                    slot = i % _NBUF
                    wait(slot)
                    lane = jnp.broadcast_to(jv[i] & 127, (16,))
                    col = jnp.broadcast_to(j0 + i, (16,))
                    for q in range(4):
                        vals = plsc.load_gather(
                            tiles_v.at[slot], [iota16 + 16 * q, lane]
                        )
                        plsc.store_scatter(
                            out_v, [iota16 + 16 * q, col], vals
                        )
                    nr = jv[i + _NBUF] if i < 16 - _NBUF else jn[i - (16 - _NBUF)]

                    @pl.when(j0 + i + _NBUF < half_w)
                    def _(nr=nr, slot=slot):
                        fetch(nr, slot)

            pltpu.sync_copy(
                out_v,
                out_hbm.at[:, pl.ds(wid * per_w + base, half_w)],
            )

    out = gather_kernel(tt, idx)
    return out.T.reshape(b, 1, _D)


# speedup vs baseline: 1.0010x; 1.0010x over previous
import functools

import jax
import jax.numpy as jnp
from jax import lax
from jax.experimental import pallas as pl
from jax.experimental.pallas import tpu as pltpu
from jax.experimental.pallas import tpu_sc as plsc

_D = 64
_NC = 2
_NS = 16
_NW = _NC * _NS
_NBUF = 4


def kernel(batch, table):
    b = batch.shape[0]
    per_w = b // _NW
    idx = batch.astype(jnp.int32).reshape(_NW, per_w)

    tt = table.T

    mesh = plsc.VectorSubcoreMesh(core_axis_name="c", subcore_axis_name="s")
    cp = pltpu.CompilerParams(
        needs_layout_passes=False, disable_bounds_checks=True
    )

    @functools.partial(
        pl.kernel,
        mesh=mesh,
        compiler_params=cp,
        out_type=jax.ShapeDtypeStruct((_D, b), table.dtype),
        scratch_types=[
            pltpu.VMEM((per_w,), jnp.int32),
            pltpu.VMEM((_NBUF, _D, 128), jnp.float32),
            pltpu.VMEM((_D, per_w // 2), jnp.float32),
            pltpu.SemaphoreType.DMA((_NBUF,)),
        ],
    )
    def gather_kernel(tt_hbm, idx_hbm, out_hbm, idx_v, tiles_v, out_v, sems):
        wid = lax.axis_index("s") * _NC + lax.axis_index("c")
        pltpu.sync_copy(idx_hbm.at[wid], idx_v)

        iota16 = lax.iota(jnp.int32, 16)
        half_w = per_w // 2

        def fetch(r, slot):
            c128 = pl.multiple_of((r >> 7) * 128, 128)
            for h in range(2):
                pltpu.async_copy(
                    tt_hbm.at[pl.ds(32 * h, 32), pl.ds(c128, 128)],
                    tiles_v.at[slot, pl.ds(32 * h, 32), :],
                    sems.at[slot],
                )

        def wait(slot):
            pltpu.make_async_copy(
                tt_hbm.at[:, pl.ds(0, 128)],
                tiles_v.at[slot],
                sems.at[slot],
            ).wait()

        for half in range(2):
            base = half * half_w
            jv0 = idx_v[pl.ds(base, 16)]
            for t in range(_NBUF):
                fetch(jv0[t], t)

            @pl.loop(0, half_w, step=16)
            def _(j0, base=base):
                jv = idx_v[pl.ds(base + j0, 16)]
                nxt = jnp.minimum(base + j0 + 16, per_w - 16)
                jn = idx_v[pl.ds(nxt, 16)]
                for i in range(16):
                    slot = i % _NBUF
                    wait(slot)
                    lane = jnp.broadcast_to(jv[i] & 127, (16,))
                    col = jnp.broadcast_to(j0 + i, (16,))
                    for q in range(4):
                        vals = plsc.load_gather(
                            tiles_v.at[slot], [iota16 + 16 * q, lane]
                        )
                        plsc.store_scatter(
                            out_v, [iota16 + 16 * q, col], vals
                        )
                    nr = jv[i + _NBUF] if i < 16 - _NBUF else jn[i - (16 - _NBUF)]

                    @pl.when(j0 + i + _NBUF < half_w)
                    def _(nr=nr, slot=slot):
                        fetch(nr, slot)

            pltpu.sync_copy(
                out_v,
                out_hbm.at[:, pl.ds(wid * per_w + base, half_w)],
            )

    out = gather_kernel(tt, idx)
    return out.T.reshape(b, 1, _D)


# speedup vs baseline: 1.0019x; 1.0009x over previous
import functools

import jax
import jax.numpy as jnp
from jax import lax
from jax.experimental import pallas as pl
from jax.experimental.pallas import tpu as pltpu
from jax.experimental.pallas import tpu_sc as plsc

_D = 64
_NC = 2
_NS = 16
_NW = _NC * _NS
_NBUF = 4


def kernel(batch, table):
    b = batch.shape[0]
    per_w = b // _NW
    idx = batch.astype(jnp.int32).reshape(_NW, per_w)

    tt = table.T

    mesh = plsc.VectorSubcoreMesh(core_axis_name="c", subcore_axis_name="s")
    cp = pltpu.CompilerParams(
        needs_layout_passes=False, disable_bounds_checks=True
    )

    @functools.partial(
        pl.kernel,
        mesh=mesh,
        compiler_params=cp,
        out_type=jax.ShapeDtypeStruct((_D, b), table.dtype),
        scratch_types=[
            pltpu.VMEM((per_w,), jnp.int32),
            pltpu.VMEM((_NBUF, _D, 128), jnp.float32),
            pltpu.VMEM((_D, per_w // 2), jnp.float32),
            pltpu.SemaphoreType.DMA((_NBUF,)),
        ],
    )
    def gather_kernel(tt_hbm, idx_hbm, out_hbm, idx_v, tiles_v, out_v, sems):
        wid = lax.axis_index("s") * _NC + lax.axis_index("c")
        pltpu.sync_copy(idx_hbm.at[wid], idx_v)

        iota16 = lax.iota(jnp.int32, 16)
        half_w = per_w // 2

        def fetch(r, slot):
            c128 = pl.multiple_of((r >> 7) * 128, 128)
            pltpu.async_copy(
                tt_hbm.at[:, pl.ds(c128, 128)],
                tiles_v.at[slot],
                sems.at[slot],
            )

        def wait(slot):
            pltpu.make_async_copy(
                tt_hbm.at[:, pl.ds(0, 128)],
                tiles_v.at[slot],
                sems.at[slot],
            ).wait()

        for half in range(2):
            base = half * half_w
            jv0 = idx_v[pl.ds(base, 16)]
            for t in range(_NBUF):
                fetch(jv0[t], t)

            @pl.loop(0, half_w, step=16)
            def _(j0, base=base):
                jv = idx_v[pl.ds(base + j0, 16)]
                nxt = jnp.minimum(base + j0 + 16, per_w - 16)
                jn = idx_v[pl.ds(nxt, 16)]
                for i in range(16):
                    slot = i % _NBUF
                    wait(slot)
                    lane = jnp.broadcast_to(jv[i] & 127, (16,))
                    col = jnp.broadcast_to(j0 + i, (16,))
                    for q in range(4):
                        vals = plsc.load_gather(
                            tiles_v.at[slot], [iota16 + 16 * q, lane]
                        )
                        plsc.store_scatter(
                            out_v, [iota16 + 16 * q, col], vals
                        )
                    nr = jv[i + _NBUF] if i < 16 - _NBUF else jn[i - (16 - _NBUF)]

                    @pl.when(j0 + i + _NBUF < half_w)
                    def _(nr=nr, slot=slot):
                        fetch(nr, slot)

            pltpu.sync_copy(
                out_v,
                out_hbm.at[:, pl.ds(wid * per_w + base, half_w)],
            )

    out = gather_kernel(tt, idx)
    return out.T.reshape(b, 1, _D)
